# baseline (speedup 1.0000x reference)
import functools

import jax
import jax.numpy as jnp
from jax.experimental import pallas as pl
from jax.experimental.pallas import tpu as pltpu

TOPK = 8


def _body(o_ref, t_ref, out_ref):
    i = pl.program_id(0)

    @pl.when(i == 0)
    def _():
        out_ref[0, 0] = 0.0

    out_ref[0, 0] += o_ref[0, 0] + t_ref[0, 0]


def kernel(output, target, target_weights):
    b, k, h, w = output.shape
    n = b * k * h * w
    o2 = output.reshape(n // 128, 128)
    t2 = target.reshape(n // 128, 128)
    rows = 6528
    nblocks = (n // 128) // rows
    f = pl.pallas_call(
        _body,
        grid=(nblocks,),
        in_specs=[
            pl.BlockSpec((rows, 128), lambda i: (i, 0)),
            pl.BlockSpec((rows, 128), lambda i: (i, 0)),
        ],
        out_specs=pl.BlockSpec((1, 1), lambda i: (0, 0), memory_space=pltpu.SMEM),
        out_shape=jax.ShapeDtypeStruct((1, 1), jnp.float32),
    )
    total = f(o2, t2)
    return (total[0, 0] / (b * TOPK) + jnp.sum(target_weights) * 0.0).astype(
        jnp.float32
    )


# speedup vs baseline: 11.0395x; 11.0395x over previous
import functools

import jax
import jax.numpy as jnp
from jax.experimental import pallas as pl
from jax.experimental.pallas import tpu as pltpu

TOPK = 8
NEG = -jnp.inf


def _body(o_ref, t_ref, tw_ref, out_ref, loss_ref, *, k, b, hw):
    i = pl.program_id(0)
    d = o_ref[0] - t_ref[0]
    s = jnp.sum(d * d, axis=(0, 1))
    loss_ref[pl.ds(i, 1), :] = s.reshape(1, b)

    @pl.when(i == k - 1)
    def _():
        tw = tw_ref[...]
        vals = loss_ref[...] * tw * tw * (1.0 / hw)
        kiota = jax.lax.broadcasted_iota(jnp.int32, (k, b), 0)
        acc = jnp.zeros((b,), jnp.float32)
        for _ in range(TOPK):
            m = jnp.max(vals, axis=0)
            acc = acc + m
            eq = vals == m[None, :]
            first = jnp.min(jnp.where(eq, kiota, k), axis=0)
            vals = jnp.where(kiota == first[None, :], NEG, vals)
        out_ref[0, 0] = jnp.sum(acc)


def kernel(output, target, target_weights):
    b, k, h, w = output.shape
    hw = h * w
    ot = jnp.transpose(output, (1, 2, 3, 0))
    tt = jnp.transpose(target, (1, 2, 3, 0))
    twt = jnp.transpose(target_weights, (1, 0))
    f = pl.pallas_call(
        functools.partial(_body, k=k, b=b, hw=hw),
        grid=(k,),
        in_specs=[
            pl.BlockSpec((1, h, w, b), lambda i: (i, 0, 0, 0)),
            pl.BlockSpec((1, h, w, b), lambda i: (i, 0, 0, 0)),
            pl.BlockSpec((k, b), lambda i: (0, 0)),
        ],
        out_specs=pl.BlockSpec((1, 1), lambda i: (0, 0), memory_space=pltpu.SMEM),
        out_shape=jax.ShapeDtypeStruct((1, 1), jnp.float32),
        scratch_shapes=[pltpu.VMEM((k, b), jnp.float32)],
    )
    total = f(ot, tt, twt)
    return (total[0, 0] / (b * TOPK)).astype(jnp.float32)
